# Initial kernel scaffold; baseline (speedup 1.0000x reference)
import functools

import jax
import jax.numpy as jnp
from jax import lax
from jax.experimental import pallas as pl
from jax.experimental.pallas import tpu as pltpu
from jax.experimental.pallas import tpu_sc as plsc

V = 1000
D = 32
N_TOTAL = 16384 * 200

NC, NS = 2, 16
NW = NC * NS
PER_W = N_TOTAL // NW
CB = 512
NCHUNK = PER_W // CB
GSZ = 128
NG = 2 * CB // GSZ

_mesh = plsc.VectorSubcoreMesh(core_axis_name="c", subcore_axis_name="s")


@functools.partial(
    pl.kernel,
    out_type=jax.ShapeDtypeStruct((2 * N_TOTAL, D), jnp.float32),
    mesh=_mesh,
    scratch_types=[
        pltpu.VMEM((CB,), jnp.int32),
        pltpu.VMEM((NG, GSZ), jnp.int32),
        pltpu.VMEM((2 * CB, D), jnp.float32),
        pltpu.SemaphoreType.DMA,
    ],
)
def _axial_kernel(idx_hbm, w_hbm, out_hbm, idx_v, c_v, rows_v, sem):
    wid = lax.axis_index("s") * NC + lax.axis_index("c")
    base0 = wid * PER_W
    lane = lax.iota(jnp.int32, 16)

    @pl.loop(0, NCHUNK)
    def _chunk(i):
        base = base0 + i * CB
        pltpu.sync_copy(idx_hbm.at[pl.ds(base, CB)], idx_v)

        @pl.loop(0, CB // 16)
        def _prep(j):
            v = idx_v[pl.ds(j * 16, 16)]
            lo = lax.rem(v, V)
            hi = lax.div(v, V) + V
            p = j * 32 + 2 * lane
            row = lax.div(p, GSZ)
            col = lax.rem(p, GSZ)
            plsc.store_scatter(c_v, [row, col], lo)
            plsc.store_scatter(c_v, [row, col + 1], hi)

        descs = [
            pltpu.async_copy(
                w_hbm.at[c_v.at[t]],
                rows_v.at[pl.ds(t * GSZ, GSZ)],
                sem,
            )
            for t in range(NG)
        ]
        for d in descs:
            d.wait()
        pltpu.sync_copy(rows_v, out_hbm.at[pl.ds(2 * base, 2 * CB)])


def kernel(idx, w0, w1):
    idx_flat = idx.reshape(-1).astype(jnp.int32)
    w = jnp.concatenate([w0, w1], axis=0)
    out = _axial_kernel(idx_flat, w)
    return out.reshape(idx.shape[0], idx.shape[1], 2 * D)


# speedup vs baseline: 8.0018x; 8.0018x over previous
import functools

import jax
import jax.numpy as jnp
from jax import lax
from jax.experimental import pallas as pl
from jax.experimental.pallas import tpu as pltpu
from jax.experimental.pallas import tpu_sc as plsc

V = 1000
D = 32
N_TOTAL = 16384 * 200

NC, NS = 2, 16
NW = NC * NS
PER_W = N_TOTAL // NW
CB = 512
NCHUNK = PER_W // CB
GSZ = 128
NG = 2 * CB // GSZ

_mesh = plsc.VectorSubcoreMesh(core_axis_name="c", subcore_axis_name="s")


@functools.partial(
    pl.kernel,
    out_type=jax.ShapeDtypeStruct((2 * N_TOTAL, D), jnp.float32),
    mesh=_mesh,
    scratch_types=[
        pltpu.VMEM((CB,), jnp.int32),
        pltpu.VMEM((2 * CB,), jnp.int32),
        pltpu.VMEM((2 * CB, D), jnp.float32),
        pltpu.SemaphoreType.DMA,
    ],
    compiler_params=pltpu.CompilerParams(
        needs_layout_passes=False, use_tc_tiling_on_sc=False
    ),
)
def _axial_kernel(idx_hbm, w_hbm, out_hbm, idx_v, c_v, rows_v, sem):
    wid = lax.axis_index("s") * NC + lax.axis_index("c")
    base0 = wid * PER_W
    lane = lax.iota(jnp.int32, 16)

    @pl.loop(0, NCHUNK)
    def _chunk(i):
        base = base0 + i * CB
        pltpu.sync_copy(idx_hbm.at[pl.ds(base, CB)], idx_v)

        @pl.loop(0, CB // 16)
        def _prep(j):
            v = idx_v[pl.ds(j * 16, 16)]
            lo = lax.rem(v, V)
            hi = lax.div(v, V) + V
            p = j * 32 + 2 * lane
            plsc.store_scatter(c_v, [p], lo)
            plsc.store_scatter(c_v, [p + 1], hi)

        descs = [
            pltpu.async_copy(
                w_hbm.at[c_v.at[pl.ds(t * GSZ, GSZ)]],
                rows_v.at[pl.ds(t * GSZ, GSZ)],
                sem,
            )
            for t in range(NG)
        ]
        for d in descs:
            d.wait()
        pltpu.sync_copy(rows_v, out_hbm.at[pl.ds(2 * base, 2 * CB)])


def kernel(idx, w0, w1):
    idx_flat = idx.reshape(-1).astype(jnp.int32)
    w = jnp.concatenate([w0, w1], axis=0)
    out = _axial_kernel(idx_flat, w)
    return out.reshape(idx.shape[0], idx.shape[1], 2 * D)


# speedup vs baseline: 8.0028x; 1.0001x over previous
import functools

import jax
import jax.numpy as jnp
from jax import lax
from jax.experimental import pallas as pl
from jax.experimental.pallas import tpu as pltpu
from jax.experimental.pallas import tpu_sc as plsc

V = 1000
D = 32
N_TOTAL = 16384 * 200

NC, NS = 2, 16
NW = NC * NS
PER_W = N_TOTAL // NW
CB = 512
NCHUNK = PER_W // CB
GSZ = 128
NG = 2 * CB // GSZ

_mesh = plsc.VectorSubcoreMesh(core_axis_name="c", subcore_axis_name="s")


@functools.partial(
    pl.kernel,
    out_type=jax.ShapeDtypeStruct((2 * N_TOTAL, D), jnp.float32),
    mesh=_mesh,
    scratch_types=[
        pltpu.VMEM((2, CB), jnp.int32),
        pltpu.VMEM((2, 2 * CB), jnp.int32),
        pltpu.VMEM((2, 2 * CB, D), jnp.float32),
        pltpu.SemaphoreType.DMA,
        pltpu.SemaphoreType.DMA,
        pltpu.SemaphoreType.DMA,
        pltpu.SemaphoreType.DMA,
        pltpu.SemaphoreType.DMA,
        pltpu.SemaphoreType.DMA,
    ],
    compiler_params=pltpu.CompilerParams(
        needs_layout_passes=False, use_tc_tiling_on_sc=False
    ),
)
def _axial_kernel(idx_hbm, w_hbm, out_hbm, idx_v, c_v, rows_v,
                  si0, si1, sg0, sg1, so0, so1):
    wid = lax.axis_index("s") * NC + lax.axis_index("c")
    base0 = wid * PER_W
    lane = lax.iota(jnp.int32, 16)
    si = (si0, si1)
    sg = (sg0, sg1)
    so = (so0, so1)

    def idx_copy(ic, b):
        return pltpu.make_async_copy(
            idx_hbm.at[pl.ds(base0 + ic * CB, CB)], idx_v.at[b], si[b]
        )

    def out_copy(ic, b):
        return pltpu.make_async_copy(
            rows_v.at[b], out_hbm.at[pl.ds(2 * (base0 + ic * CB), 2 * CB)], so[b]
        )

    idx_copy(0, 0).start()

    @pl.loop(0, NCHUNK, step=2)
    def _chunk(i):
        for b in (0, 1):
            ic = i + b

            @pl.when(ic >= 2)
            def _():
                out_copy(ic - 2, b).wait()

            idx_copy(ic, b).wait()

            @pl.loop(0, CB // 16)
            def _prep(j):
                v = idx_v[b, pl.ds(j * 16, 16)]
                lo = lax.rem(v, V)
                hi = lax.div(v, V) + V
                p = j * 32 + 2 * lane
                plsc.store_scatter(c_v.at[b], [p], lo)
                plsc.store_scatter(c_v.at[b], [p + 1], hi)

            descs = [
                pltpu.async_copy(
                    w_hbm.at[c_v.at[b, pl.ds(t * GSZ, GSZ)]],
                    rows_v.at[b, pl.ds(t * GSZ, GSZ)],
                    sg[b],
                )
                for t in range(NG)
            ]

            @pl.when(ic + 1 < NCHUNK)
            def _():
                idx_copy(ic + 1, 1 - b).start()

            for d in descs:
                d.wait()
            out_copy(ic, b).start()

    out_copy(NCHUNK - 2, 0).wait()
    out_copy(NCHUNK - 1, 1).wait()


def kernel(idx, w0, w1):
    idx_flat = idx.reshape(-1).astype(jnp.int32)
    w = jnp.concatenate([w0, w1], axis=0)
    out = _axial_kernel(idx_flat, w)
    return out.reshape(idx.shape[0], idx.shape[1], 2 * D)
